# baseline (speedup 1.0000x reference)
import functools

import jax
import jax.numpy as jnp
from jax import lax
from jax.experimental import pallas as pl
from jax.experimental.pallas import tpu as pltpu
from jax.experimental.pallas import tpu_sc as plsc


BT = 256

_SC_CORES = 2
_SC_SUBCORES = 16
_NW = _SC_CORES * _SC_SUBCORES


def _sc_permute_rows(table, src_idx, dst_idx, n_rows_out, row_chunk, nbuf=4):
    D = table.shape[1]
    n_idx = src_idx.shape[0]
    rows_per_w = n_idx // _NW
    nch = rows_per_w // row_chunk
    assert n_idx % (_NW * row_chunk) == 0 and nch % nbuf == 0
    assert row_chunk % 8 == 0
    src3 = src_idx.reshape(_NW, nch, row_chunk)
    dst3 = dst_idx.reshape(_NW, nch, row_chunk)
    mesh = plsc.VectorSubcoreMesh(core_axis_name="c", subcore_axis_name="s")

    @functools.partial(
        pl.kernel,
        mesh=mesh,
        out_type=jax.ShapeDtypeStruct((n_rows_out, D), table.dtype),
        scratch_types=[
            pltpu.VMEM((nch, row_chunk), jnp.int32),
            pltpu.VMEM((nch, row_chunk), jnp.int32),
        ]
        + [pltpu.VMEM((row_chunk, D), table.dtype) for _ in range(nbuf)]
        + [pltpu.SemaphoreType.DMA for _ in range(2 * nbuf)],
    )
    def permute_kernel(tab_hbm, src_hbm, dst_hbm, out_hbm, sidx_v, didx_v, *scratch):
        bufs = scratch[:nbuf]
        gsems = scratch[nbuf : 2 * nbuf]
        wsems = scratch[2 * nbuf :]
        wid = lax.axis_index("s") * _SC_CORES + lax.axis_index("c")
        pltpu.sync_copy(src_hbm.at[wid], sidx_v)
        pltpu.sync_copy(dst_hbm.at[wid], didx_v)

        def start_gather(chunk, j):
            pltpu.async_copy(tab_hbm.at[sidx_v.at[chunk]], bufs[j], gsems[j])

        def wait_gather(chunk, j):
            pltpu.make_async_copy(
                tab_hbm.at[sidx_v.at[chunk]], bufs[j], gsems[j]
            ).wait()

        def start_scatter(chunk, j):
            pltpu.async_copy(bufs[j], out_hbm.at[didx_v.at[chunk]], wsems[j])

        def wait_scatter(chunk, j):
            pltpu.make_async_copy(
                bufs[j], out_hbm.at[didx_v.at[chunk]], wsems[j]
            ).wait()

        for j in range(nbuf):
            start_gather(j, j)

        @pl.loop(0, nch, step=nbuf)
        def _(i):
            for j in range(nbuf):
                c = i + j
                wait_gather(c, j)
                start_scatter(c, j)

                @pl.when(c + nbuf < nch)
                def _(c=c, j=j):
                    wait_scatter(c, j)
                    start_gather(c + nbuf, j)

            @pl.when(i + nbuf >= nch)
            def _(i=i):
                for j in range(nbuf):
                    wait_scatter(i + j, j)

    return permute_kernel(table, src3, dst3)


def _routing_metadata(top_k_index, n_exp, n_tok, k):
    S = n_tok * k
    NB = S // BT + n_exp
    NPAD = NB * BT

    e_flat = top_k_index.reshape(-1).astype(jnp.int32)
    eye = jnp.arange(n_exp, dtype=jnp.int32)
    onehot = (e_flat[:, None] == eye[None, :]).astype(jnp.int32)
    csum = jnp.cumsum(onehot, axis=0)
    counts = csum[-1]
    rank = jnp.sum((csum - 1) * onehot, axis=1)
    nb_e = (counts + BT - 1) // BT
    cum_nb = jnp.cumsum(nb_e)
    blk_start = cum_nb - nb_e
    dest = jnp.sum(onehot * blk_start[None, :], axis=1) * BT + rank
    block_expert = jnp.sum(
        (cum_nb[None, :] <= jnp.arange(NB, dtype=jnp.int32)[:, None]).astype(
            jnp.int32
        ),
        axis=1,
    )
    block_expert = jnp.minimum(block_expert, n_exp - 1)
    return dest.astype(jnp.int32), block_expert.astype(jnp.int32), NB, NPAD


def _expert_changed(be_ref):
    b = pl.program_id(0)
    return (b == 0) | (be_ref[b] != be_ref[jnp.maximum(b - 1, 0)])


def _swiglu_body(I, be_ref, x_ref, w_ref, act_ref, wbf_ref):
    @pl.when(_expert_changed(be_ref))
    def _():
        wbf_ref[...] = w_ref[0].astype(jnp.bfloat16)

    gu = jax.lax.dot_general(
        x_ref[...], wbf_ref[...], (((1,), (1,)), ((), ())),
        preferred_element_type=jnp.float32,
    )
    g = gu[:, :I]
    u = gu[:, I:]
    act_ref[...] = ((g * jax.nn.sigmoid(g)) * u).astype(jnp.bfloat16)


def _down_body(be_ref, act_ref, w_ref, y_ref, wbf_ref):
    @pl.when(_expert_changed(be_ref))
    def _():
        wbf_ref[...] = w_ref[0].astype(jnp.bfloat16)

    y_ref[...] = jax.lax.dot_general(
        act_ref[...], wbf_ref[...], (((1,), (1,)), ((), ())),
        preferred_element_type=jnp.float32,
    )


def kernel(hidden_states, top_k_index, top_k_weights, gate_up_proj, down_proj):
    n_tok, H = hidden_states.shape
    n_exp, twoI, _ = gate_up_proj.shape
    I = twoI // 2
    k = top_k_index.shape[1]
    S = n_tok * k

    dest, block_expert, NB, NPAD = _routing_metadata(top_k_index, n_exp, n_tok, k)

    hid_bf = hidden_states.astype(jnp.bfloat16)
    hid_i32 = jax.lax.bitcast_convert_type(
        hid_bf.reshape(n_tok, H // 2, 2), jnp.int32
    )
    t_flat = (jnp.arange(S, dtype=jnp.int32) // k)
    xs_i32 = _sc_permute_rows(hid_i32, t_flat, dest, NPAD, 16)
    x_sorted = jax.lax.bitcast_convert_type(xs_i32, jnp.bfloat16).reshape(
        NPAD, H
    )

    cparams = pltpu.CompilerParams(
        dimension_semantics=("arbitrary",),
        vmem_limit_bytes=100 * 1024 * 1024,
    )

    act = pl.pallas_call(
        functools.partial(_swiglu_body, I),
        grid_spec=pltpu.PrefetchScalarGridSpec(
            num_scalar_prefetch=1,
            grid=(NB,),
            in_specs=[
                pl.BlockSpec((BT, H), lambda b, be: (b, 0)),
                pl.BlockSpec((1, twoI, H), lambda b, be: (be[b], 0, 0)),
            ],
            out_specs=pl.BlockSpec((BT, I), lambda b, be: (b, 0)),
            scratch_shapes=[pltpu.VMEM((twoI, H), jnp.bfloat16)],
        ),
        out_shape=jax.ShapeDtypeStruct((NPAD, I), jnp.bfloat16),
        compiler_params=cparams,
    )(block_expert, x_sorted, gate_up_proj)

    y = pl.pallas_call(
        _down_body,
        grid_spec=pltpu.PrefetchScalarGridSpec(
            num_scalar_prefetch=1,
            grid=(NB,),
            in_specs=[
                pl.BlockSpec((BT, I), lambda b, be: (b, 0)),
                pl.BlockSpec((1, H, I), lambda b, be: (be[b], 0, 0)),
            ],
            out_specs=pl.BlockSpec((BT, H), lambda b, be: (b, 0)),
            scratch_shapes=[pltpu.VMEM((H, I), jnp.bfloat16)],
        ),
        out_shape=jax.ShapeDtypeStruct((NPAD, H), jnp.float32),
        compiler_params=cparams,
    )(block_expert, act, down_proj)

    dest2 = dest.reshape(n_tok, k)
    out = top_k_weights[:, 0:1] * jnp.take(y, dest2[:, 0], axis=0)
    out = out + top_k_weights[:, 1:2] * jnp.take(y, dest2[:, 1], axis=0)
    return out


# speedup vs baseline: 1.9943x; 1.9943x over previous
import functools

import jax
import jax.numpy as jnp
from jax import lax
from jax.experimental import pallas as pl
from jax.experimental.pallas import tpu as pltpu
from jax.experimental.pallas import tpu_sc as plsc


BT = 256

_SC_CORES = 2
_SC_SUBCORES = 16
_NW = _SC_CORES * _SC_SUBCORES


def _sc_permute_rows(table, src_idx, dst_idx, n_rows_out, row_chunk, nbuf=4):
    D = table.shape[1]
    n_idx = src_idx.shape[0]
    rows_per_w = n_idx // _NW
    nch = rows_per_w // row_chunk
    assert n_idx % (_NW * row_chunk) == 0 and nch % nbuf == 0
    assert row_chunk % 8 == 0
    src3 = src_idx.reshape(_NW, nch, row_chunk)
    dst3 = dst_idx.reshape(_NW, nch, row_chunk)
    mesh = plsc.VectorSubcoreMesh(core_axis_name="c", subcore_axis_name="s")

    @functools.partial(
        pl.kernel,
        mesh=mesh,
        out_type=jax.ShapeDtypeStruct((n_rows_out, D), table.dtype),
        scratch_types=[
            pltpu.VMEM((nch, row_chunk), jnp.int32),
            pltpu.VMEM((nch, row_chunk), jnp.int32),
        ]
        + [pltpu.VMEM((row_chunk, D), table.dtype) for _ in range(nbuf)]
        + [pltpu.SemaphoreType.DMA for _ in range(2 * nbuf)],
    )
    def permute_kernel(tab_hbm, src_hbm, dst_hbm, out_hbm, sidx_v, didx_v, *scratch):
        bufs = scratch[:nbuf]
        gsems = scratch[nbuf : 2 * nbuf]
        wsems = scratch[2 * nbuf :]
        wid = lax.axis_index("s") * _SC_CORES + lax.axis_index("c")
        pltpu.sync_copy(src_hbm.at[wid], sidx_v)
        pltpu.sync_copy(dst_hbm.at[wid], didx_v)

        def start_gather(chunk, j):
            pltpu.async_copy(tab_hbm.at[sidx_v.at[chunk]], bufs[j], gsems[j])

        def wait_gather(chunk, j):
            pltpu.make_async_copy(
                tab_hbm.at[sidx_v.at[chunk]], bufs[j], gsems[j]
            ).wait()

        def start_scatter(chunk, j):
            pltpu.async_copy(bufs[j], out_hbm.at[didx_v.at[chunk]], wsems[j])

        def wait_scatter(chunk, j):
            pltpu.make_async_copy(
                bufs[j], out_hbm.at[didx_v.at[chunk]], wsems[j]
            ).wait()

        for j in range(nbuf):
            start_gather(j, j)

        @pl.loop(0, nch, step=nbuf)
        def _(i):
            for j in range(nbuf):
                c = i + j
                wait_gather(c, j)
                start_scatter(c, j)

                @pl.when(c + nbuf < nch)
                def _(c=c, j=j):
                    wait_scatter(c, j)
                    start_gather(c + nbuf, j)

            @pl.when(i + nbuf >= nch)
            def _(i=i):
                for j in range(nbuf):
                    wait_scatter(i + j, j)

    return permute_kernel(table, src3, dst3)


def _routing_metadata(top_k_index, n_exp, n_tok, k):
    S = n_tok * k
    NB = S // BT + n_exp
    NPAD = NB * BT

    e_flat = top_k_index.reshape(-1).astype(jnp.int32)
    eye = jnp.arange(n_exp, dtype=jnp.int32)
    onehot = (e_flat[:, None] == eye[None, :]).astype(jnp.int32)
    csum = jnp.cumsum(onehot, axis=0)
    counts = csum[-1]
    rank = jnp.sum((csum - 1) * onehot, axis=1)
    nb_e = (counts + BT - 1) // BT
    cum_nb = jnp.cumsum(nb_e)
    blk_start = cum_nb - nb_e
    dest = jnp.sum(onehot * blk_start[None, :], axis=1) * BT + rank
    block_expert = jnp.sum(
        (cum_nb[None, :] <= jnp.arange(NB, dtype=jnp.int32)[:, None]).astype(
            jnp.int32
        ),
        axis=1,
    )
    block_expert = jnp.minimum(block_expert, n_exp - 1)
    return dest.astype(jnp.int32), block_expert.astype(jnp.int32), NB, NPAD


def _expert_changed(be_ref):
    b = pl.program_id(0)
    return (b == 0) | (be_ref[b] != be_ref[jnp.maximum(b - 1, 0)])


def _swiglu_body(I, be_ref, x_ref, w_ref, act_ref, wbf_ref):
    @pl.when(_expert_changed(be_ref))
    def _():
        wbf_ref[...] = w_ref[0].astype(jnp.bfloat16)

    gu = jax.lax.dot_general(
        x_ref[...], wbf_ref[...], (((1,), (1,)), ((), ())),
        preferred_element_type=jnp.float32,
    )
    g = gu[:, :I]
    u = gu[:, I:]
    act_ref[...] = ((g * jax.nn.sigmoid(g)) * u).astype(jnp.bfloat16)


def _down_body(be_ref, act_ref, w_ref, y_ref, wbf_ref):
    @pl.when(_expert_changed(be_ref))
    def _():
        wbf_ref[...] = w_ref[0].astype(jnp.bfloat16)

    y_ref[...] = jax.lax.dot_general(
        act_ref[...], wbf_ref[...], (((1,), (1,)), ((), ())),
        preferred_element_type=jnp.float32,
    )


def kernel(hidden_states, top_k_index, top_k_weights, gate_up_proj, down_proj):
    n_tok, H = hidden_states.shape
    n_exp, twoI, _ = gate_up_proj.shape
    I = twoI // 2
    k = top_k_index.shape[1]
    S = n_tok * k

    dest, block_expert, NB, NPAD = _routing_metadata(top_k_index, n_exp, n_tok, k)

    t_flat = (jnp.arange(S, dtype=jnp.int32) // k)
    x_sorted = _sc_permute_rows(hidden_states, t_flat, dest, NPAD, 8)
    x_sorted = x_sorted.astype(jnp.bfloat16)

    cparams = pltpu.CompilerParams(
        dimension_semantics=("arbitrary",),
        vmem_limit_bytes=100 * 1024 * 1024,
    )

    act = pl.pallas_call(
        functools.partial(_swiglu_body, I),
        grid_spec=pltpu.PrefetchScalarGridSpec(
            num_scalar_prefetch=1,
            grid=(NB,),
            in_specs=[
                pl.BlockSpec((BT, H), lambda b, be: (b, 0)),
                pl.BlockSpec((1, twoI, H), lambda b, be: (be[b], 0, 0)),
            ],
            out_specs=pl.BlockSpec((BT, I), lambda b, be: (b, 0)),
            scratch_shapes=[pltpu.VMEM((twoI, H), jnp.bfloat16)],
        ),
        out_shape=jax.ShapeDtypeStruct((NPAD, I), jnp.bfloat16),
        compiler_params=cparams,
    )(block_expert, x_sorted, gate_up_proj)

    y = pl.pallas_call(
        _down_body,
        grid_spec=pltpu.PrefetchScalarGridSpec(
            num_scalar_prefetch=1,
            grid=(NB,),
            in_specs=[
                pl.BlockSpec((BT, I), lambda b, be: (b, 0)),
                pl.BlockSpec((1, H, I), lambda b, be: (be[b], 0, 0)),
            ],
            out_specs=pl.BlockSpec((BT, H), lambda b, be: (b, 0)),
            scratch_shapes=[pltpu.VMEM((H, I), jnp.bfloat16)],
        ),
        out_shape=jax.ShapeDtypeStruct((NPAD, H), jnp.float32),
        compiler_params=cparams,
    )(block_expert, act, down_proj)

    dest2 = dest.reshape(n_tok, k)
    out = top_k_weights[:, 0:1] * jnp.take(y, dest2[:, 0], axis=0)
    out = out + top_k_weights[:, 1:2] * jnp.take(y, dest2[:, 1], axis=0)
    return out


# speedup vs baseline: 2.2514x; 1.1289x over previous
import functools

import jax
import jax.numpy as jnp
from jax import lax
from jax.experimental import pallas as pl
from jax.experimental.pallas import tpu as pltpu
from jax.experimental.pallas import tpu_sc as plsc


BT = 256

_SC_CORES = 2
_SC_SUBCORES = 16
_NW = _SC_CORES * _SC_SUBCORES


def _sc_permute_rows(table, src_idx, dst_idx, n_rows_out, row_chunk, nbuf=4):
    D = table.shape[1]
    n_idx = src_idx.shape[0]
    rows_per_w = n_idx // _NW
    nch = rows_per_w // row_chunk
    assert n_idx % (_NW * row_chunk) == 0 and nch % nbuf == 0
    assert row_chunk % 8 == 0
    src3 = src_idx.reshape(_NW, nch, row_chunk)
    dst3 = dst_idx.reshape(_NW, nch, row_chunk)
    mesh = plsc.VectorSubcoreMesh(core_axis_name="c", subcore_axis_name="s")

    @functools.partial(
        pl.kernel,
        mesh=mesh,
        out_type=jax.ShapeDtypeStruct((n_rows_out, D), table.dtype),
        scratch_types=[
            pltpu.VMEM((nch, row_chunk), jnp.int32),
            pltpu.VMEM((nch, row_chunk), jnp.int32),
        ]
        + [pltpu.VMEM((row_chunk, D), table.dtype) for _ in range(nbuf)]
        + [pltpu.SemaphoreType.DMA for _ in range(2 * nbuf)],
    )
    def permute_kernel(tab_hbm, src_hbm, dst_hbm, out_hbm, sidx_v, didx_v, *scratch):
        bufs = scratch[:nbuf]
        gsems = scratch[nbuf : 2 * nbuf]
        wsems = scratch[2 * nbuf :]
        wid = lax.axis_index("s") * _SC_CORES + lax.axis_index("c")
        pltpu.sync_copy(src_hbm.at[wid], sidx_v)
        pltpu.sync_copy(dst_hbm.at[wid], didx_v)

        def start_gather(chunk, j):
            pltpu.async_copy(tab_hbm.at[sidx_v.at[chunk]], bufs[j], gsems[j])

        def wait_gather(chunk, j):
            pltpu.make_async_copy(
                tab_hbm.at[sidx_v.at[chunk]], bufs[j], gsems[j]
            ).wait()

        def start_scatter(chunk, j):
            pltpu.async_copy(bufs[j], out_hbm.at[didx_v.at[chunk]], wsems[j])

        def wait_scatter(chunk, j):
            pltpu.make_async_copy(
                bufs[j], out_hbm.at[didx_v.at[chunk]], wsems[j]
            ).wait()

        for j in range(nbuf):
            start_gather(j, j)

        @pl.loop(0, nch, step=nbuf)
        def _(i):
            for j in range(nbuf):
                c = i + j
                wait_gather(c, j)
                start_scatter(c, j)

                @pl.when(c + nbuf < nch)
                def _(c=c, j=j):
                    wait_scatter(c, j)
                    start_gather(c + nbuf, j)

            @pl.when(i + nbuf >= nch)
            def _(i=i):
                for j in range(nbuf):
                    wait_scatter(i + j, j)

    return permute_kernel(table, src3, dst3)


def _routing_metadata(top_k_index, n_exp, n_tok, k):
    S = n_tok * k
    NB = S // BT + n_exp
    NPAD = NB * BT

    e_flat = top_k_index.reshape(-1).astype(jnp.int32)
    eye = jnp.arange(n_exp, dtype=jnp.int32)
    onehot = (e_flat[:, None] == eye[None, :]).astype(jnp.int32)
    csum = jnp.cumsum(onehot, axis=0)
    counts = csum[-1]
    rank = jnp.sum((csum - 1) * onehot, axis=1)
    nb_e = (counts + BT - 1) // BT
    cum_nb = jnp.cumsum(nb_e)
    blk_start = cum_nb - nb_e
    dest = jnp.sum(onehot * blk_start[None, :], axis=1) * BT + rank
    block_expert = jnp.sum(
        (cum_nb[None, :] <= jnp.arange(NB, dtype=jnp.int32)[:, None]).astype(
            jnp.int32
        ),
        axis=1,
    )
    block_expert = jnp.minimum(block_expert, n_exp - 1)
    be_all = jnp.concatenate([block_expert, cum_nb[-1:]])
    return dest.astype(jnp.int32), be_all.astype(jnp.int32), NB, NPAD


def _swiglu_body(I, NB, be_ref, x_ref, w_ref, act_ref):
    @pl.when(pl.program_id(0) < be_ref[NB])
    def _():
        x = x_ref[...].astype(jnp.bfloat16)
        w = w_ref[0].astype(jnp.bfloat16)
        gu = jax.lax.dot_general(
            x, w, (((1,), (1,)), ((), ())), preferred_element_type=jnp.float32
        )
        g = gu[:, :I]
        u = gu[:, I:]
        act_ref[...] = (g * jax.nn.sigmoid(g)) * u


def _down_body(NB, be_ref, act_ref, w_ref, y_ref):
    @pl.when(pl.program_id(0) < be_ref[NB])
    def _():
        a = act_ref[...].astype(jnp.bfloat16)
        w = w_ref[0].astype(jnp.bfloat16)
        y_ref[...] = jax.lax.dot_general(
            a, w, (((1,), (1,)), ((), ())), preferred_element_type=jnp.float32
        )


def kernel(hidden_states, top_k_index, top_k_weights, gate_up_proj, down_proj):
    n_tok, H = hidden_states.shape
    n_exp, twoI, _ = gate_up_proj.shape
    I = twoI // 2
    k = top_k_index.shape[1]
    S = n_tok * k

    dest, block_expert, NB, NPAD = _routing_metadata(top_k_index, n_exp, n_tok, k)

    t_flat = (jnp.arange(S, dtype=jnp.int32) // k)
    x_sorted = _sc_permute_rows(hidden_states, t_flat, dest, NPAD, 8)

    cparams = pltpu.CompilerParams(
        dimension_semantics=("arbitrary",),
        vmem_limit_bytes=100 * 1024 * 1024,
    )

    def row_map(b, be):
        return (jnp.minimum(b, be[NB] - 1), 0)

    act = pl.pallas_call(
        functools.partial(_swiglu_body, I, NB),
        grid_spec=pltpu.PrefetchScalarGridSpec(
            num_scalar_prefetch=1,
            grid=(NB,),
            in_specs=[
                pl.BlockSpec((BT, H), row_map),
                pl.BlockSpec((1, twoI, H), lambda b, be: (be[b], 0, 0)),
            ],
            out_specs=pl.BlockSpec((BT, I), lambda b, be: (b, 0)),
        ),
        out_shape=jax.ShapeDtypeStruct((NPAD, I), jnp.float32),
        compiler_params=cparams,
    )(block_expert, x_sorted, gate_up_proj)

    y = pl.pallas_call(
        functools.partial(_down_body, NB),
        grid_spec=pltpu.PrefetchScalarGridSpec(
            num_scalar_prefetch=1,
            grid=(NB,),
            in_specs=[
                pl.BlockSpec((BT, I), row_map),
                pl.BlockSpec((1, H, I), lambda b, be: (be[b], 0, 0)),
            ],
            out_specs=pl.BlockSpec((BT, H), lambda b, be: (b, 0)),
        ),
        out_shape=jax.ShapeDtypeStruct((NPAD, H), jnp.float32),
        compiler_params=cparams,
    )(block_expert, act, down_proj)

    dest2 = dest.reshape(n_tok, k)
    out = top_k_weights[:, 0:1] * jnp.take(y, dest2[:, 0], axis=0)
    out = out + top_k_weights[:, 1:2] * jnp.take(y, dest2[:, 1], axis=0)
    return out
